# baseline (speedup 1.0000x reference)
import jax
import jax.numpy as jnp
from jax import lax
from jax.experimental import pallas as pl
from jax.experimental.pallas import tpu as pltpu
from jax.experimental.pallas import tpu_sc as plsc

N = 100000
ROW = 384
NC = 2
NS = 16
NW = NC * NS
L = 16
WPR = 3136
CH = 112
NCHUNK = WPR // CH
NG = CH // L


def _compute_chunk(buf, tmp, outv, off):
    iota = lax.iota(jnp.int32, L)
    idx_base = iota * L

    def group_body(g, carry):
        base = g * L
        for r16 in range(L):
            r = base + r16
            acc = None
            for j in range(8):
                s = buf[r, pl.ds(j * L, L)]
                p = buf[r, pl.ds(128 + j * L, L)]
                o = buf[r, pl.ds(256 + j * L, L)]
                prod = s * p * o
                acc = prod if acc is None else acc + prod
            tmp[pl.ds(r16 * L, L)] = acc
        colsum = None
        for c in range(L):
            v = plsc.load_gather(tmp, [idx_base + c])
            colsum = v if colsum is None else colsum + v
        outv[pl.ds(off + base, L)] = colsum
        return carry

    lax.fori_loop(0, NG, group_body, 0)


def _body(x_hbm, out_hbm, buf0, buf1, tmp, outv, sem0, sem1):
    wid = lax.axis_index("s") * NC + lax.axis_index("c")
    start = jnp.minimum(wid * WPR, N - WPR)

    def src(ci):
        return x_hbm.at[pl.ds(start + ci * CH, CH)]

    pltpu.async_copy(src(0), buf0, sem0)

    def pair_body(i, carry):
        ci = 2 * i
        pltpu.async_copy(src(ci + 1), buf1, sem1)
        pltpu.make_async_copy(src(ci), buf0, sem0).wait()
        _compute_chunk(buf0, tmp, outv, ci * CH)
        pltpu.async_copy(src(ci + 2), buf0, sem0)
        pltpu.make_async_copy(src(ci + 1), buf1, sem1).wait()
        _compute_chunk(buf1, tmp, outv, (ci + 1) * CH)
        return carry

    lax.fori_loop(0, NCHUNK // 2 - 1, pair_body, 0)

    ci = NCHUNK - 2
    pltpu.async_copy(src(ci + 1), buf1, sem1)
    pltpu.make_async_copy(src(ci), buf0, sem0).wait()
    _compute_chunk(buf0, tmp, outv, ci * CH)
    pltpu.make_async_copy(src(ci + 1), buf1, sem1).wait()
    _compute_chunk(buf1, tmp, outv, (ci + 1) * CH)

    pltpu.sync_copy(outv, out_hbm.at[pl.ds(start, WPR)])


@jax.jit
def kernel(triples):
    x = triples.reshape(N, ROW)
    k = pl.kernel(
        _body,
        out_type=jax.ShapeDtypeStruct((N,), jnp.float32),
        mesh=plsc.VectorSubcoreMesh(core_axis_name="c", subcore_axis_name="s"),
        scratch_types=[
            pltpu.VMEM((CH, ROW), jnp.float32),
            pltpu.VMEM((CH, ROW), jnp.float32),
            pltpu.VMEM((L * L,), jnp.float32),
            pltpu.VMEM((WPR,), jnp.float32),
            pltpu.SemaphoreType.DMA,
            pltpu.SemaphoreType.DMA,
        ],
        compiler_params=pltpu.CompilerParams(needs_layout_passes=False),
    )
    return k(x)
